# Initial kernel scaffold; baseline (speedup 1.0000x reference)
import jax
import jax.numpy as jnp
from jax.experimental import pallas as pl


def kernel(features, edges, edge_weights, params):
    raise NotImplementedError("write your pallas kernel here")


# speedup vs baseline: 13.1767x; 13.1767x over previous
import functools

import jax
import jax.numpy as jnp
from jax import lax
from jax.experimental import pallas as pl
from jax.experimental.pallas import tpu as pltpu
from jax.experimental.pallas import tpu_sc as plsc

_BN_EPS = 1e-3
_SQRT_HALF = 0.7071067811865476

_NC, _NS, _L = 2, 16, 16
_NW = _NC * _NS
_B = 80
_CR = 5
_H = 32
_HP = 40


def _gelu(x):
    return x * 0.5 * (1.0 + lax.erf(x * _SQRT_HALF))


def _fold_layer(p):
    scale = p["gamma"] / jnp.sqrt(p["var"] + _BN_EPS)
    shift = p["beta"] - p["mean"] * scale
    w = scale[:, None] * p["W"]
    b = shift @ p["W"] + p["b"]
    return w, b


def _tc1_body(f_ref, w1_ref, b1_ref, w2_ref, b2_ref, ua_ref, u1_ref,
              msgs_ref, part_ref):
    x = f_ref[...]
    n = x.shape[0]
    h1 = _gelu(jnp.dot(x, w1_ref[...], preferred_element_type=jnp.float32)
               + b1_ref[...])
    m = _gelu(jnp.dot(h1, w2_ref[...], preferred_element_type=jnp.float32)
              + b2_ref[...])
    col = lax.broadcasted_iota(jnp.int32, (n, _HP - _H), 1)
    tail = jnp.where(col == 0, 1.0, 0.0).astype(jnp.float32)
    msgs_ref[...] = jnp.concatenate([m, tail], axis=1)
    part_ref[...] = (jnp.dot(x, ua_ref[...], preferred_element_type=jnp.float32)
                     + u1_ref[...])


def _tc2_body(part_ref, acc_ref, ub_ref, u2_ref, b2_ref, out_ref):
    n = part_ref.shape[0]
    t = (acc_ref[0] + acc_ref[1])[:n]
    s = t[:, :_H]
    c = t[:, _H:_H + 1]
    agg = s / jnp.maximum(c, 1.0)
    x1 = _gelu(part_ref[...]
               + jnp.dot(agg, ub_ref[...], preferred_element_type=jnp.float32))
    x2 = _gelu(jnp.dot(x1, u2_ref[...], preferred_element_type=jnp.float32)
               + b2_ref[...])
    ss = jnp.sum(x2 * x2, axis=-1, keepdims=True)
    out_ref[...] = x2 * lax.rsqrt(jnp.maximum(ss, 1e-12))


def _splat16(vec, j):
    return lax.gather(
        vec, jnp.full((_L, 1), j, jnp.int32),
        lax.GatherDimensionNumbers(offset_dims=(), collapsed_slice_dims=(0,),
                                   start_index_map=(0,)),
        (1,), mode=lax.GatherScatterMode.PROMISE_IN_BOUNDS)


@functools.lru_cache(maxsize=None)
def _make_sc_edge(np_, e):
    rows_tile = e // _NW // _B
    nch = rows_tile // _CR
    rpt = np_ // _NS

    @functools.partial(
        pl.kernel,
        out_type=jax.ShapeDtypeStruct((_NC, np_, _HP), jnp.float32),
        mesh=plsc.VectorSubcoreMesh(core_axis_name="c", subcore_axis_name="s"),
        compiler_params=pltpu.CompilerParams(use_tc_tiling_on_sc=False),
        scratch_types=[
            pltpu.VMEM((128,), jnp.int32),
            pltpu.VMEM((128, _B), jnp.int32),
            pltpu.VMEM((128, _B), jnp.int32),
            pltpu.VMEM((128, _B), jnp.float32),
            pltpu.VMEM((_CR * _B, _HP), jnp.float32),
            pltpu.VMEM_SHARED((np_, _HP), jnp.float32),
            pltpu.SemaphoreType.DMA,
        ])
    def sc_edge(msgs_hbm, src_hbm, dst_hbm, w_hbm, iota_hbm, zero_hbm,
                acc_out,
                iidx_v, src_v, dst_v, w_v, rows_v,
                acc_sh, sem):
        cid = lax.axis_index("c")
        sid = lax.axis_index("s")
        wid = cid * _NS + sid

        r0 = sid * rpt
        pltpu.sync_copy(zero_hbm, acc_sh.at[pl.ds(r0, rpt)])
        pltpu.sync_copy(iota_hbm.at[wid], iidx_v)
        pltpu.async_copy(src_hbm.at[iidx_v], src_v, sem).wait()
        pltpu.async_copy(dst_hbm.at[iidx_v], dst_v, sem).wait()
        pltpu.async_copy(w_hbm.at[iidx_v], w_v, sem).wait()
        plsc.subcore_barrier()

        for k in range(nch):
            cps = [pltpu.async_copy(msgs_hbm.at[src_v.at[k * _CR + j]],
                                    rows_v.at[pl.ds(j * _B, _B)], sem)
                   for j in range(_CR)]
            for cp in cps:
                cp.wait()

            def gbody(j, carry):
                for s5 in range(_B // _L):
                    w16 = w_v[k * _CR + j, pl.ds(s5 * _L, _L)]
                    for l in range(_L):
                        ws = _splat16(w16, l)
                        r = j * _B + s5 * _L + l
                        rows_v[r, pl.ds(0, _L)] = rows_v[r, pl.ds(0, _L)] * ws
                        rows_v[r, pl.ds(_L, _L)] = (
                            rows_v[r, pl.ds(_L, _L)] * ws)
                return carry
            lax.fori_loop(0, _CR, gbody, 0)

            for j in range(_CR):
                pltpu.sync_copy(rows_v.at[pl.ds(j * _B, _B)],
                                acc_sh.at[dst_v.at[k * _CR + j]], add=True)

        plsc.subcore_barrier()
        pltpu.sync_copy(acc_sh.at[pl.ds(r0, rpt)],
                        acc_out.at[cid, pl.ds(r0, rpt)])

    return sc_edge


def kernel(features, edges, edge_weights, params):
    n, d = features.shape
    e = edges.shape[1]
    np_ = ((n + 16 * 8 - 1) // (16 * 8)) * (16 * 8)

    w1, b1 = _fold_layer(params["prepare"][0])
    w2, b2 = _fold_layer(params["prepare"][1])
    uw1, ub1 = _fold_layer(params["update"][0])
    uw2, ub2 = _fold_layer(params["update"][1])
    ua, ub = uw1[:d], uw1[d:]

    msgs, part = pl.pallas_call(
        _tc1_body,
        out_shape=[jax.ShapeDtypeStruct((n, _HP), jnp.float32),
                   jax.ShapeDtypeStruct((n, _H), jnp.float32)],
    )(features, w1, b1[None], w2, b2[None], ua, ub1[None])

    msgs_p = jnp.pad(msgs, ((0, np_ - n), (0, 0)))
    rows = e // _B
    rows_tile = rows // _NW
    src2d = edges[1].reshape(rows, _B)
    dst2d = edges[0].reshape(rows, _B)
    w2d = edge_weights.reshape(rows, _B)
    iota = jnp.pad(
        jnp.arange(rows, dtype=jnp.int32).reshape(_NW, rows_tile),
        ((0, 0), (0, 128 - rows_tile)))
    zeros = jnp.zeros((np_ // _NS, _HP), jnp.float32)
    acc = _make_sc_edge(np_, e)(msgs_p, src2d, dst2d, w2d, iota, zeros)

    out = pl.pallas_call(
        _tc2_body,
        out_shape=jax.ShapeDtypeStruct((n, _H), jnp.float32),
    )(part, acc, ub, uw2, ub2[None])
    return out
